# Initial kernel scaffold; baseline (speedup 1.0000x reference)
import jax
import jax.numpy as jnp
from jax.experimental import pallas as pl


def kernel(x, time_day, time_week):
    raise NotImplementedError("write your pallas kernel here")


# speedup vs baseline: 6.7680x; 6.7680x over previous
import functools

import jax
import jax.numpy as jnp
from jax import lax
from jax.experimental import pallas as pl
from jax.experimental.pallas import tpu as pltpu
from jax.experimental.pallas import tpu_sc as plsc

_TIME = 288
_B, _T, _N, _C = 64, 12, 8192, 3
_F = 64
_L = 16
_NC, _NS = 2, 16
_NW = _NC * _NS
_NPW = _N // _NW
_NVEC = _NPW // _L
_WPAD = 8


def _tec_body(day_hbm, week_hbm, dayt_hbm, weekt_hbm, out_hbm,
              stage_v, didx_v, widx_v, dayt_v, weekt_v, outbuf_v):
    cid = lax.axis_index("c")
    sid = lax.axis_index("s")
    wid = sid * _NC + cid
    n0 = wid * _NPW

    pltpu.sync_copy(dayt_hbm, dayt_v)
    pltpu.sync_copy(weekt_hbm, weekt_v)

    pltpu.sync_copy(day_hbm.at[:, pl.ds(n0, _NPW)], stage_v)

    def day_idx_body(i, _):
        b = i // _NVEC
        j = i - b * _NVEC
        v = stage_v[b, pl.ds(j * _L, _L)]
        d = lax.convert_element_type(v * float(_TIME), jnp.int32)
        didx_v[b, pl.ds(j * _L, _L)] = jnp.clip(d, 0, _TIME - 1)
        return 0

    lax.fori_loop(0, _B * _NVEC, day_idx_body, 0)

    pltpu.sync_copy(week_hbm.at[:, pl.ds(n0, _NPW)], stage_v)

    def week_idx_body(i, _):
        b = i // _NVEC
        j = i - b * _NVEC
        v = stage_v[b, pl.ds(j * _L, _L)]
        w = lax.convert_element_type(v, jnp.int32)
        widx_v[b, pl.ds(j * _L, _L)] = jnp.clip(w, 0, 6)
        return 0

    lax.fori_loop(0, _B * _NVEC, week_idx_body, 0)

    def batch_body(b, _):
        def vec_body(j, _):
            dvec = didx_v[b, pl.ds(j * _L, _L)]
            wvec = widx_v[b, pl.ds(j * _L, _L)]
            for f in range(_F):
                vd = plsc.load_gather(dayt_v, [dvec + (f * _TIME)])
                vw = plsc.load_gather(weekt_v, [wvec + (f * _WPAD)])
                outbuf_v[f, pl.ds(j * _L, _L)] = vd + vw
            return 0

        lax.fori_loop(0, _NVEC, vec_body, 0)
        pltpu.sync_copy(outbuf_v, out_hbm.at[b, :, pl.ds(n0, _NPW)])
        return 0

    lax.fori_loop(0, _B, batch_body, 0)


@functools.partial(
    pl.kernel,
    mesh=plsc.VectorSubcoreMesh(core_axis_name="c", subcore_axis_name="s"),
    out_type=jax.ShapeDtypeStruct((_B, _F, _N), jnp.float32),
    compiler_params=pltpu.CompilerParams(needs_layout_passes=False),
    scratch_types=[
        pltpu.VMEM((_B, _NPW), jnp.float32),
        pltpu.VMEM((_B, _NPW), jnp.int32),
        pltpu.VMEM((_B, _NPW), jnp.int32),
        pltpu.VMEM((_F * _TIME,), jnp.float32),
        pltpu.VMEM((_F * _WPAD,), jnp.float32),
        pltpu.VMEM((_F, _NPW), jnp.float32),
    ],
)
def _sc_lookup(day_hbm, week_hbm, dayt_hbm, weekt_hbm, out_hbm,
               stage_v, didx_v, widx_v, dayt_v, weekt_v, outbuf_v):
    _tec_body(day_hbm, week_hbm, dayt_hbm, weekt_hbm, out_hbm,
              stage_v, didx_v, widx_v, dayt_v, weekt_v, outbuf_v)


def kernel(x, time_day, time_week):
    day_frac = x[:, _T - 1, :, 1]
    week_val = x[:, _T - 1, :, 2]
    dayt = jnp.transpose(time_day).reshape(-1)
    weekt = jnp.concatenate(
        [jnp.transpose(time_week),
         jnp.zeros((_F, _WPAD - 7), jnp.float32)], axis=1).reshape(-1)
    out = _sc_lookup(day_frac, week_val, dayt, weekt)
    return out[..., None]
